# baseline (speedup 1.0000x reference)
import functools

import jax
import jax.numpy as jnp
from jax import lax
from jax.experimental import pallas as pl
from jax.experimental.pallas import tpu as pltpu
from jax.experimental.pallas import tpu_sc as plsc

_NSUB = 16
_CH = 128


def _sc_segment_sum(feat2, edge_index, zeros_blk, *, n_feat, n_acc, n_chunks):
    zr = n_acc // _NSUB
    cps_lo = n_chunks // _NSUB
    rem = n_chunks % _NSUB
    max_pairs = (cps_lo + 2) // 2
    mesh = plsc.VectorSubcoreMesh(core_axis_name="c", subcore_axis_name="s")

    @functools.partial(
        pl.kernel,
        out_type=jax.ShapeDtypeStruct((2 * n_acc, 128), jnp.float32),
        mesh=mesh,
        scratch_types=[
            pltpu.VMEM((2, 2, _CH), jnp.int32),
            pltpu.VMEM((2, _CH, 128), jnp.float32),
            pltpu.VMEM_SHARED((n_acc, 128), jnp.float32),
            pltpu.SemaphoreType.DMA,
            pltpu.SemaphoreType.DMA,
            pltpu.SemaphoreType.DMA,
            pltpu.SemaphoreType.DMA,
        ],
    )
    def body(feat_hbm, e_hbm, zero_hbm, out_hbm, idx, rows, acc,
             gsem0, gsem1, isem0, isem1):
        c = lax.axis_index("c")
        s = lax.axis_index("s")
        gsems = (gsem0, gsem1)
        isems = (isem0, isem1)
        row0 = s * zr
        cnt = cps_lo + jnp.where(s < rem, 1, 0)
        base = s * cps_lo + jnp.minimum(s, rem)

        def iload(b, ck):
            pltpu.async_copy(e_hbm.at[:, pl.ds((base + ck) * _CH, _CH)],
                             idx.at[b], isems[b])

        def iwait(b, ck):
            pltpu.make_async_copy(e_hbm.at[:, pl.ds((base + ck) * _CH, _CH)],
                                  idx.at[b], isems[b]).wait()

        def fixup(b):
            srow = idx.at[b].at[0]
            for i in range(_CH // 16):
                sl = pl.ds(i * 16, 16)
                srow[sl] = srow[sl] * 2 + c

        def gstart(b, ck):
            pltpu.async_copy(feat_hbm.at[idx.at[b].at[0]], rows.at[b],
                             gsems[b])

        def gwait(b):
            pltpu.make_async_copy(feat_hbm.at[idx.at[b].at[0]], rows.at[b],
                                  gsems[b]).wait()

        iload(0, 0)
        iload(1, 1)
        pltpu.sync_copy(zero_hbm, acc.at[pl.ds(row0, zr)])
        iwait(0, 0)
        fixup(0)
        gstart(0, 0)
        plsc.subcore_barrier()

        @pl.loop(0, max_pairs)
        def _(j):
            k = 2 * j
            for b in range(2):
                ck = k + b
                b2 = 1 - b

                @pl.when(ck < cnt)
                def _():
                    gwait(b)

                    @pl.when(ck + 1 < cnt)
                    def _():
                        iwait(b2, ck + 1)
                        fixup(b2)
                        gstart(b2, ck + 1)

                    pltpu.sync_copy(rows.at[b], acc.at[idx.at[b].at[1]],
                                    add=True)

                    @pl.when(ck + 2 < cnt)
                    def _():
                        iload(b, ck + 2)

        plsc.subcore_barrier()
        pltpu.sync_copy(acc.at[pl.ds(row0, zr)],
                        out_hbm.at[pl.ds(c * n_acc + row0, zr)])

    return body(feat2, edge_index, zeros_blk)


def _tc_mlp(feat, neigh3, W1b, b1, W2b, b2, eps, *, bm):
    n, d = feat.shape
    h = W1b.shape[1]
    nb = n // bm

    def body(eps_sm, feat_r, n3_r, w1_r, b1_r, w2_r, b2_r, out_r):
        scale = 1.0 + eps_sm[0]
        neigh = jnp.concatenate([n3_r[0], n3_r[1]], axis=1)
        rst = scale * feat_r[...].astype(jnp.float32) + neigh
        acts = jnp.maximum(
            jnp.dot(rst.astype(jnp.bfloat16), w1_r[...],
                    preferred_element_type=jnp.float32) + b1_r[...], 0.0)
        out_r[...] = (jnp.dot(acts.astype(jnp.bfloat16), w2_r[...],
                              preferred_element_type=jnp.float32) + b2_r[...])

    return pl.pallas_call(
        body,
        grid=(nb,),
        in_specs=[
            pl.BlockSpec(memory_space=pltpu.SMEM),
            pl.BlockSpec((bm, d), lambda i: (i, 0)),
            pl.BlockSpec((2, bm, 128), lambda i: (0, i, 0)),
            pl.BlockSpec((d, h), lambda i: (0, 0)),
            pl.BlockSpec((1, h), lambda i: (0, 0)),
            pl.BlockSpec((h, d), lambda i: (0, 0)),
            pl.BlockSpec((1, d), lambda i: (0, 0)),
        ],
        out_specs=pl.BlockSpec((bm, d), lambda i: (i, 0)),
        out_shape=jax.ShapeDtypeStruct((n, d), jnp.float32),
        compiler_params=pltpu.CompilerParams(
            dimension_semantics=("arbitrary",)),
    )(eps, feat, neigh3, W1b,
      b1.reshape(1, h), W2b, b2.reshape(1, d))


def kernel(feat, edge_index, W1, b1, W2, b2, eps):
    n, d = feat.shape
    e = edge_index.shape[1]
    bm = 5000
    n_acc = 10240
    n_chunks = e // _CH

    feat2 = feat.reshape(2 * n, 128)
    zeros_blk = jnp.zeros((n_acc // _NSUB, 128), jnp.float32)

    neigh_flat = _sc_segment_sum(feat2, edge_index, zeros_blk,
                                 n_feat=n, n_acc=n_acc, n_chunks=n_chunks)
    return _tc_mlp(feat.astype(jnp.bfloat16), neigh_flat.reshape(2, n_acc, 128),
                   W1.astype(jnp.bfloat16), b1,
                   W2.astype(jnp.bfloat16), b2, eps, bm=bm)


# speedup vs baseline: 1.0039x; 1.0039x over previous
import functools

import jax
import jax.numpy as jnp
from jax import lax
from jax.experimental import pallas as pl
from jax.experimental.pallas import tpu as pltpu
from jax.experimental.pallas import tpu_sc as plsc

_NSUB = 16
_CH = 128


def _sc_segment_sum(feat2, edge_index, zeros_blk, *, n_feat, n_acc, n_chunks):
    zr = n_acc // _NSUB
    cps_lo = n_chunks // _NSUB
    rem = n_chunks % _NSUB
    max_pairs = (cps_lo + 2) // 2
    mesh = plsc.VectorSubcoreMesh(core_axis_name="c", subcore_axis_name="s")

    @functools.partial(
        pl.kernel,
        out_type=jax.ShapeDtypeStruct((2 * n_acc, 128), jnp.float32),
        mesh=mesh,
        scratch_types=[
            pltpu.VMEM((2, 2, _CH), jnp.int32),
            pltpu.VMEM((2, _CH, 128), jnp.float32),
            pltpu.VMEM_SHARED((n_acc, 128), jnp.float32),
            pltpu.SemaphoreType.DMA,
            pltpu.SemaphoreType.DMA,
            pltpu.SemaphoreType.DMA,
            pltpu.SemaphoreType.DMA,
        ],
    )
    def body(feat_hbm, e_hbm, zero_hbm, out_hbm, idx, rows, acc,
             gsem0, gsem1, isem0, isem1):
        c = lax.axis_index("c")
        s = lax.axis_index("s")
        gsems = (gsem0, gsem1)
        isems = (isem0, isem1)
        row0 = s * zr
        cnt = cps_lo + jnp.where(s < rem, 1, 0)
        base = s * cps_lo + jnp.minimum(s, rem)

        def iload(b, ck):
            pltpu.async_copy(e_hbm.at[:, pl.ds((base + ck) * _CH, _CH)],
                             idx.at[b], isems[b])

        def iwait(b, ck):
            pltpu.make_async_copy(e_hbm.at[:, pl.ds((base + ck) * _CH, _CH)],
                                  idx.at[b], isems[b]).wait()

        def fixup(b):
            srow = idx.at[b].at[0]
            for i in range(_CH // 16):
                sl = pl.ds(i * 16, 16)
                srow[sl] = srow[sl] * 2 + c

        def gstart(b, ck):
            pltpu.async_copy(feat_hbm.at[idx.at[b].at[0]], rows.at[b],
                             gsems[b])

        def gwait(b):
            pltpu.make_async_copy(feat_hbm.at[idx.at[b].at[0]], rows.at[b],
                                  gsems[b]).wait()

        iload(0, 0)
        iload(1, 1)
        pltpu.sync_copy(zero_hbm, acc.at[pl.ds(row0, zr)])
        iwait(0, 0)
        fixup(0)
        gstart(0, 0)
        plsc.subcore_barrier()

        @pl.loop(0, max_pairs)
        def _(j):
            k = 2 * j
            for b in range(2):
                ck = k + b
                b2 = 1 - b

                @pl.when(ck < cnt)
                def _():
                    gwait(b)

                    @pl.when(ck + 1 < cnt)
                    def _():
                        iwait(b2, ck + 1)
                        fixup(b2)
                        gstart(b2, ck + 1)

                    pltpu.sync_copy(rows.at[b], acc.at[idx.at[b].at[1]],
                                    add=True)

                    @pl.when(ck + 2 < cnt)
                    def _():
                        iload(b, ck + 2)

        plsc.subcore_barrier()
        pltpu.sync_copy(acc.at[pl.ds(row0, zr)],
                        out_hbm.at[pl.ds(c * n_acc + row0, zr)])

    return body(feat2, edge_index, zeros_blk)


def _tc_mlp(feat, neigh3, W1b, b1, W2b, b2, eps, *, bm):
    n, d = feat.shape
    h = W1b.shape[1]
    nb = n // bm

    def body(eps_sm, feat_r, n3_r, w1_r, b1_r, w2_r, b2_r, out_r):
        scale = 1.0 + eps_sm[0]
        neigh = jnp.concatenate([n3_r[0], n3_r[1]], axis=1)
        rst = scale * feat_r[...].astype(jnp.float32) + neigh
        acts = jnp.maximum(
            jnp.dot(rst.astype(jnp.bfloat16), w1_r[...],
                    preferred_element_type=jnp.float32) + b1_r[...], 0.0)
        out_r[...] = (jnp.dot(acts.astype(jnp.bfloat16), w2_r[...],
                              preferred_element_type=jnp.float32) + b2_r[...])

    return pl.pallas_call(
        body,
        grid=(nb,),
        in_specs=[
            pl.BlockSpec(memory_space=pltpu.SMEM),
            pl.BlockSpec((bm, d), lambda i: (i, 0)),
            pl.BlockSpec((2, bm, 128), lambda i: (0, i, 0)),
            pl.BlockSpec((d, h), lambda i: (0, 0)),
            pl.BlockSpec((1, h), lambda i: (0, 0)),
            pl.BlockSpec((h, d), lambda i: (0, 0)),
            pl.BlockSpec((1, d), lambda i: (0, 0)),
        ],
        out_specs=pl.BlockSpec((bm, d), lambda i: (i, 0)),
        out_shape=jax.ShapeDtypeStruct((n, d), jnp.float32),
        compiler_params=pltpu.CompilerParams(
            dimension_semantics=("arbitrary",)),
    )(eps, feat, neigh3, W1b,
      b1.reshape(1, h), W2b, b2.reshape(1, d))


def kernel(feat, edge_index, W1, b1, W2, b2, eps):
    n, d = feat.shape
    e = edge_index.shape[1]
    bm = 2000
    n_acc = 10240
    n_chunks = e // _CH

    feat2 = feat.reshape(2 * n, 128)
    zeros_blk = jnp.zeros((n_acc // _NSUB, 128), jnp.float32)

    neigh_flat = _sc_segment_sum(feat2, edge_index, zeros_blk,
                                 n_feat=n, n_acc=n_acc, n_chunks=n_chunks)
    return _tc_mlp(feat.astype(jnp.bfloat16), neigh_flat.reshape(2, n_acc, 128),
                   W1.astype(jnp.bfloat16), b1,
                   W2.astype(jnp.bfloat16), b2, eps, bm=bm)
